# baseline (speedup 1.0000x reference)
import functools

import jax
import jax.numpy as jnp
from jax import lax
from jax.experimental import pallas as pl
from jax.experimental.pallas import tpu as pltpu
from jax.experimental.pallas import tpu_sc as plsc

_NC = 2
_NS = 16
_CHUNK = 80
_EPS = 1e-5


def _silu(v):
    return v * jax.nn.sigmoid(v)


def _pack2(p1, p3):
    bf = jnp.bfloat16
    f32 = jnp.float32
    i1 = lax.bitcast_convert_type(p1.astype(bf).astype(f32), jnp.int32)
    i3 = lax.bitcast_convert_type(p3.astype(bf).astype(f32), jnp.int32)
    return i3 | lax.shift_right_logical(i1, 16)


def _unpack2(w):
    f32 = jnp.float32
    p1 = lax.bitcast_convert_type(lax.shift_left(w, 16), f32)
    p3 = lax.bitcast_convert_type(w & jnp.int32(-65536), f32)
    return p1, p3


def _proj_body(fc, x_ref, w1_ref, w3_ref, outa_ref, outb_ref):
    bf = jnp.bfloat16
    f32 = jnp.float32
    xb = x_ref[...].astype(bf)
    w1 = w1_ref[...].astype(bf)
    w3 = w3_ref[...].astype(bf)
    pa1 = jnp.dot(xb, w1[:fc], preferred_element_type=f32)
    pa3 = jnp.dot(xb, w3[:fc], preferred_element_type=f32)
    pb1 = jnp.dot(xb, w1[fc:], preferred_element_type=f32)
    pb3 = jnp.dot(xb, w3[fc:], preferred_element_type=f32)
    outa_ref[...] = _pack2(pa1, pa3)
    outb_ref[...] = _pack2(pb1, pb3)


def _gather_body(n, fc, nchunk, ept_dir, tbla_hbm, tblb_hbm, idx_hbm,
                 out_hbm, shared_tbl, ib0, ib1, ib2, ib3, db0, db1, db2, db3,
                 i0, i1, i2, i3, g0, g1, g2, g3, s0, s1, s2, s3):
    cid = lax.axis_index("c")
    sid = lax.axis_index("s")
    nzc = n // _CHUNK
    iters = (nzc + _NS - 1) // _NS

    def stage(k, carry):
        c = k * _NS + sid

        @pl.when(c < nzc)
        def _():
            @pl.when(cid == 0)
            def _():
                pltpu.sync_copy(tblb_hbm.at[pl.ds(c * _CHUNK, _CHUNK)], db0)

            @pl.when(cid == 1)
            def _():
                pltpu.sync_copy(tbla_hbm.at[pl.ds(c * _CHUNK, _CHUNK)], db0)

            pltpu.sync_copy(db0, shared_tbl.at[pl.ds(c * _CHUNK, _CHUNK)])

        return carry

    lax.fori_loop(0, iters, stage, 0)
    plsc.subcore_barrier()

    ib = (ib0, ib1, ib2, ib3)
    db = (db0, db1, db2, db3)
    isem = (i0, i1, i2, i3)
    gsem = (g0, g1, g2, g3)
    ssem = (s0, s1, s2, s3)
    nset = 4
    niter = nchunk // nset
    base = sid * ept_dir

    def drain_store(k):
        pltpu.make_async_copy(
            db[k], out_hbm.at[cid, pl.ds(base, _CHUNK)], ssem[k]).wait()

    def body(i, carry):
        his = []
        for k in range(nset):
            c = nset * i + k

            @pl.when(i > 0)
            def _(k=k):
                drain_store(k)

            his.append(pltpu.async_copy(
                idx_hbm.at[cid, sid, c], ib[k], isem[k]))
        hgs = []
        for k in range(nset):
            his[k].wait()
            hgs.append(pltpu.async_copy(shared_tbl.at[ib[k]], db[k], gsem[k]))
        for k in range(nset):
            c = nset * i + k
            hgs[k].wait()
            pltpu.async_copy(
                db[k], out_hbm.at[cid, pl.ds(base + c * _CHUNK, _CHUNK)],
                ssem[k])
        return carry

    lax.fori_loop(0, niter, body, 0)
    for k in range(nset):
        drain_store(k)
    for c in range(niter * nset, nchunk):
        pltpu.sync_copy(idx_hbm.at[cid, sid, c], ib[0])
        hg = pltpu.async_copy(shared_tbl.at[ib[0]], db[0], g0)
        hg.wait()
        pltpu.sync_copy(
            db[0], out_hbm.at[cid, pl.ds(base + c * _CHUNK, _CHUNK)])


def _mlp_body(fc, neb, pi_ref, pj_ref, ea_ref, w1c_ref, w3c_ref, w2_ref,
              w4_ref, b1_ref, b2_ref, b3_ref, b4_ref, z_ref, m_ref, zs_ref,
              zq_ref, acc_s, acc_q):
    i = pl.program_id(0)
    bf = jnp.bfloat16
    f32 = jnp.float32
    pi1, pi3 = _unpack2(pi_ref[0])
    pj1, pj3 = _unpack2(pj_ref[0])
    ea = ea_ref[...].astype(bf)
    w2 = w2_ref[...].astype(bf)
    w4 = w4_ref[...].astype(bf)

    a1 = (pi1 + pj1
          + jnp.dot(ea, w1c_ref[...].astype(bf), preferred_element_type=f32)
          + b1_ref[...])
    z = jnp.dot(_silu(a1).astype(bf), w2, preferred_element_type=f32) \
        + b2_ref[...]
    a3 = (pi3 + pj3
          + jnp.dot(ea, w3c_ref[...].astype(bf), preferred_element_type=f32)
          + b3_ref[...])
    m = jnp.dot(_silu(a3).astype(bf), w4, preferred_element_type=f32) \
        + b4_ref[...]

    z_ref[...] = z.astype(bf)
    m_ref[...] = m.astype(bf)

    @pl.when(i == 0)
    def _():
        acc_s[...] = jnp.zeros_like(acc_s)
        acc_q[...] = jnp.zeros_like(acc_q)

    acc_s[...] += jnp.sum(z, axis=0, keepdims=True)
    acc_q[...] += jnp.sum(z * z, axis=0, keepdims=True)

    @pl.when(i == neb - 1)
    def _():
        zs_ref[...] = acc_s[...]
        zq_ref[...] = acc_q[...]


def _score_body(e_f, z_ref, m_ref, zsa_ref, zqa_ref, zsb_ref, zqb_ref,
                g_ref, b_ref, msg_ref):
    mu = (zsa_ref[...] + zsb_ref[...]) / e_f
    var = (zqa_ref[...] + zqb_ref[...]) / e_f - mu * mu
    rstd = lax.rsqrt(var + _EPS)
    z = z_ref[...].astype(jnp.float32)
    zn = (z - mu) * (rstd * g_ref[...]) + b_ref[...]
    score = jax.nn.sigmoid(zn)
    msg_ref[...] = score * m_ref[...].astype(jnp.float32)


def _scatter_body(n, fc, nchunk, ept, rch, msg_hbm, idx_hbm, out_hbm,
                  shared, vbuf, idxs, m0, m1, l0, l1, t0, t1):
    cid = lax.axis_index("c")
    sid = lax.axis_index("s")
    wid = cid * _NS + sid
    nzc = n // rch
    iters = (nzc + _NS - 1) // _NS

    zero16 = jnp.zeros((16,), jnp.float32)
    lanes = fc // 16

    def zb(k, carry):
        vbuf[k // lanes, pl.ds((k % lanes) * 16, 16)] = zero16
        return carry

    lax.fori_loop(0, rch * lanes, zb, 0)

    def zcopy(k, carry):
        c = k * _NS + sid

        @pl.when(c < nzc)
        def _():
            pltpu.sync_copy(vbuf, shared.at[pl.ds(c * rch, rch)])

        return carry

    lax.fori_loop(0, iters, zcopy, 0)
    plsc.subcore_barrier()

    pltpu.sync_copy(idx_hbm.at[wid], idxs)
    ebase = wid * ept
    mb = (m0, m1)
    ls = (l0, l1)
    ts = (t0, t1)
    nset = 2
    niter = nchunk // nset

    def drain_scatter(k):
        pltpu.make_async_copy(mb[k], shared.at[idxs.at[0]], ts[k]).wait()

    def sb(i, carry):
        hs = []
        for k in range(nset):
            c = nset * i + k

            @pl.when(i > 0)
            def _(k=k):
                drain_scatter(k)

            hs.append(pltpu.async_copy(
                msg_hbm.at[pl.ds(ebase + c * _CHUNK, _CHUNK)], mb[k], ls[k]))
        for k in range(nset):
            c = nset * i + k
            hs[k].wait()
            pltpu.async_copy(mb[k], shared.at[idxs.at[c]], ts[k], add=True)
        return carry

    lax.fori_loop(0, niter, sb, 0)
    for k in range(nset):
        drain_scatter(k)
    for c in range(niter * nset, nchunk):
        pltpu.sync_copy(msg_hbm.at[pl.ds(ebase + c * _CHUNK, _CHUNK)], mb[0])
        pltpu.sync_copy(mb[0], shared.at[idxs.at[c]], add=True)
    plsc.subcore_barrier()

    def ob(k, carry):
        c = k * _NS + sid

        @pl.when(c < nzc)
        def _():
            pltpu.sync_copy(shared.at[pl.ds(c * rch, rch)], vbuf)
            pltpu.sync_copy(vbuf, out_hbm.at[cid, pl.ds(c * rch, rch)])

        return carry

    lax.fori_loop(0, iters, ob, 0)


def _final_body(n_f, p0a_ref, p1a_ref, p0b_ref, p1b_ref, x_ref, g_ref,
                b_ref, y_ref, acc_s, acc_q):
    ph = pl.program_id(0)
    i = pl.program_id(1)
    o = (p0a_ref[0] + p1a_ref[0]) + (p0b_ref[0] + p1b_ref[0])

    @pl.when(ph == 0)
    def _():
        @pl.when(i == 0)
        def _():
            acc_s[...] = jnp.zeros_like(acc_s)
            acc_q[...] = jnp.zeros_like(acc_q)

        acc_s[...] += jnp.sum(o, axis=0, keepdims=True)
        acc_q[...] += jnp.sum(o * o, axis=0, keepdims=True)

    @pl.when(ph == 1)
    def _():
        mu = acc_s[...] / n_f
        var = acc_q[...] / n_f - mu * mu
        rstd = lax.rsqrt(var + _EPS)
        y_ref[...] = jnp.maximum(
            x_ref[...] + (o - mu) * (rstd * g_ref[...]) + b_ref[...], 0.0)


def kernel(x, edge_index, edge_attr, W1, b1, W2, b2, W3, b3, W4, b4,
           g_int, b_int, g_bn, b_bn):
    f32 = jnp.float32
    bf = jnp.bfloat16
    n, fc = x.shape
    e = edge_index.shape[1]
    nw = _NC * _NS
    nb = 2000
    nb0 = 5000
    rch = 80
    e_a = 163840
    halves = ((0, e_a, 8192), (e_a, e - e_a, 7808))

    src = edge_index[0]
    dst = edge_index[1]

    tbl_a, tbl_b = pl.pallas_call(
        functools.partial(_proj_body, fc),
        grid=(n // nb0,),
        in_specs=[
            pl.BlockSpec((nb0, fc), lambda i: (i, 0)),
            pl.BlockSpec((2 * fc, fc), lambda i: (0, 0)),
            pl.BlockSpec((2 * fc, fc), lambda i: (0, 0)),
        ],
        out_specs=[
            pl.BlockSpec((nb0, fc), lambda i: (i, 0)),
            pl.BlockSpec((nb0, fc), lambda i: (i, 0)),
        ],
        out_shape=[
            jax.ShapeDtypeStruct((n, fc), jnp.int32),
            jax.ShapeDtypeStruct((n, fc), jnp.int32),
        ],
    )(x, W1[:2 * fc], W3[:2 * fc])

    mesh = plsc.VectorSubcoreMesh(core_axis_name="c", subcore_axis_name="s")

    zs, qs, zm, dst3s, sizes = [], [], [], [], []
    for (off, e_h, eb) in halves:
        ept = e_h // nw
        nchunk = ept // _CHUNK
        ept_dir = e_h // _NS
        nchunk_dir = ept_dir // _CHUNK
        neb = e_h // eb
        src_h = lax.dynamic_slice_in_dim(src, off, e_h)
        dst_h = lax.dynamic_slice_in_dim(dst, off, e_h)
        ea_h = lax.dynamic_slice_in_dim(edge_attr, off, e_h)
        idx_g = jnp.stack([src_h, dst_h]).reshape(2, _NS, nchunk_dir, _CHUNK)
        dst3s.append(dst_h.reshape(nw, nchunk, _CHUNK))
        sizes.append((e_h, ept, nchunk, neb, eb))

        gathered = pl.kernel(
            functools.partial(_gather_body, n, fc, nchunk_dir, ept_dir),
            out_type=jax.ShapeDtypeStruct((2, e_h, fc), jnp.int32),
            mesh=mesh,
            scratch_types=(
                [pltpu.VMEM_SHARED((n, fc), jnp.int32)]
                + [pltpu.VMEM((_CHUNK,), jnp.int32) for _ in range(4)]
                + [pltpu.VMEM((_CHUNK, fc), jnp.int32) for _ in range(4)]
                + [pltpu.SemaphoreType.DMA for _ in range(12)]
            ),
        )(tbl_a, tbl_b, idx_g)

        z, m, zsum, zsq = pl.pallas_call(
            functools.partial(_mlp_body, fc, neb),
            grid=(neb,),
            in_specs=[
                pl.BlockSpec((1, eb, fc), lambda i: (1, i, 0)),
                pl.BlockSpec((1, eb, fc), lambda i: (0, i, 0)),
                pl.BlockSpec((eb, fc), lambda i: (i, 0)),
                pl.BlockSpec((fc, fc), lambda i: (0, 0)),
                pl.BlockSpec((fc, fc), lambda i: (0, 0)),
                pl.BlockSpec((fc, fc), lambda i: (0, 0)),
                pl.BlockSpec((fc, fc), lambda i: (0, 0)),
                pl.BlockSpec((1, fc), lambda i: (0, 0)),
                pl.BlockSpec((1, fc), lambda i: (0, 0)),
                pl.BlockSpec((1, fc), lambda i: (0, 0)),
                pl.BlockSpec((1, fc), lambda i: (0, 0)),
            ],
            out_specs=[
                pl.BlockSpec((eb, fc), lambda i: (i, 0)),
                pl.BlockSpec((eb, fc), lambda i: (i, 0)),
                pl.BlockSpec((1, fc), lambda i: (0, 0)),
                pl.BlockSpec((1, fc), lambda i: (0, 0)),
            ],
            out_shape=[
                jax.ShapeDtypeStruct((e_h, fc), bf),
                jax.ShapeDtypeStruct((e_h, fc), bf),
                jax.ShapeDtypeStruct((1, fc), f32),
                jax.ShapeDtypeStruct((1, fc), f32),
            ],
            scratch_shapes=[pltpu.VMEM((1, fc), f32),
                            pltpu.VMEM((1, fc), f32)],
        )(gathered, gathered, ea_h, W1[2 * fc:], W3[2 * fc:], W2, W4,
          b1.reshape(1, fc), b2.reshape(1, fc), b3.reshape(1, fc),
          b4.reshape(1, fc))
        zm.append((z, m))
        zs.append(zsum)
        qs.append(zsq)

    partials = []
    for h in range(2):
        e_h, ept, nchunk, neb, eb = sizes[h]
        z, m = zm[h]
        msg = pl.pallas_call(
            functools.partial(_score_body, float(e)),
            grid=(neb,),
            in_specs=[
                pl.BlockSpec((eb, fc), lambda i: (i, 0)),
                pl.BlockSpec((eb, fc), lambda i: (i, 0)),
                pl.BlockSpec((1, fc), lambda i: (0, 0)),
                pl.BlockSpec((1, fc), lambda i: (0, 0)),
                pl.BlockSpec((1, fc), lambda i: (0, 0)),
                pl.BlockSpec((1, fc), lambda i: (0, 0)),
                pl.BlockSpec((1, fc), lambda i: (0, 0)),
                pl.BlockSpec((1, fc), lambda i: (0, 0)),
            ],
            out_specs=pl.BlockSpec((eb, fc), lambda i: (i, 0)),
            out_shape=jax.ShapeDtypeStruct((e_h, fc), f32),
        )(z, m, zs[0], qs[0], zs[1], qs[1],
          g_int.reshape(1, fc), b_int.reshape(1, fc))

        partials.append(pl.kernel(
            functools.partial(_scatter_body, n, fc, nchunk, ept, rch),
            out_type=jax.ShapeDtypeStruct((_NC, n, fc), f32),
            mesh=mesh,
            scratch_types=(
                [pltpu.VMEM_SHARED((n, fc), f32),
                 pltpu.VMEM((rch, fc), f32),
                 pltpu.VMEM((nchunk, _CHUNK), jnp.int32)]
                + [pltpu.VMEM((_CHUNK, fc), f32) for _ in range(2)]
                + [pltpu.SemaphoreType.DMA for _ in range(4)]
            ),
        )(msg, dst3s[h]))

    y = pl.pallas_call(
        functools.partial(_final_body, float(n)),
        grid=(2, n // nb),
        in_specs=[
            pl.BlockSpec((1, nb, fc), lambda p, i: (0, i, 0)),
            pl.BlockSpec((1, nb, fc), lambda p, i: (1, i, 0)),
            pl.BlockSpec((1, nb, fc), lambda p, i: (0, i, 0)),
            pl.BlockSpec((1, nb, fc), lambda p, i: (1, i, 0)),
            pl.BlockSpec((nb, fc), lambda p, i: (i, 0)),
            pl.BlockSpec((1, fc), lambda p, i: (0, 0)),
            pl.BlockSpec((1, fc), lambda p, i: (0, 0)),
        ],
        out_specs=pl.BlockSpec((nb, fc), lambda p, i: (i, 0)),
        out_shape=jax.ShapeDtypeStruct((n, fc), f32),
        scratch_shapes=[pltpu.VMEM((1, fc), f32), pltpu.VMEM((1, fc), f32)],
    )(partials[0], partials[0], partials[1], partials[1], x,
      g_bn.reshape(1, fc), b_bn.reshape(1, fc))

    return y


# speedup vs baseline: 1.0768x; 1.0768x over previous
import functools

import jax
import jax.numpy as jnp
from jax import lax
from jax.experimental import pallas as pl
from jax.experimental.pallas import tpu as pltpu
from jax.experimental.pallas import tpu_sc as plsc

_NC = 2
_NS = 16
_CHUNK = 80
_EPS = 1e-5


def _silu(v):
    return v * jax.nn.sigmoid(v)


def _pack2(p1, p3):
    bf = jnp.bfloat16
    f32 = jnp.float32
    i1 = lax.bitcast_convert_type(p1.astype(bf).astype(f32), jnp.int32)
    i3 = lax.bitcast_convert_type(p3.astype(bf).astype(f32), jnp.int32)
    return i3 | lax.shift_right_logical(i1, 16)


def _unpack2(w):
    f32 = jnp.float32
    p1 = lax.bitcast_convert_type(lax.shift_left(w, 16), f32)
    p3 = lax.bitcast_convert_type(w & jnp.int32(-65536), f32)
    return p1, p3


def _proj_body(fc, x_ref, w1_ref, w3_ref, outa_ref, outb_ref):
    bf = jnp.bfloat16
    f32 = jnp.float32
    xb = x_ref[...].astype(bf)
    w1 = w1_ref[...].astype(bf)
    w3 = w3_ref[...].astype(bf)
    pa1 = jnp.dot(xb, w1[:fc], preferred_element_type=f32)
    pa3 = jnp.dot(xb, w3[:fc], preferred_element_type=f32)
    pb1 = jnp.dot(xb, w1[fc:], preferred_element_type=f32)
    pb3 = jnp.dot(xb, w3[fc:], preferred_element_type=f32)
    outa_ref[...] = _pack2(pa1, pa3)
    outb_ref[...] = _pack2(pb1, pb3)


def _gather_body(n, fc, nchunk, ept_dir, tbla_hbm, tblb_hbm, idx_hbm,
                 out_hbm, shared_tbl, ib0, ib1, ib2, ib3, db0, db1, db2, db3,
                 i0, i1, i2, i3, g0, g1, g2, g3, s0, s1, s2, s3):
    cid = lax.axis_index("c")
    sid = lax.axis_index("s")
    nzc = n // _CHUNK
    iters = (nzc + _NS - 1) // _NS

    def stage(k, carry):
        c = k * _NS + sid

        @pl.when(c < nzc)
        def _():
            @pl.when(cid == 0)
            def _():
                pltpu.sync_copy(tblb_hbm.at[pl.ds(c * _CHUNK, _CHUNK)], db0)

            @pl.when(cid == 1)
            def _():
                pltpu.sync_copy(tbla_hbm.at[pl.ds(c * _CHUNK, _CHUNK)], db0)

            pltpu.sync_copy(db0, shared_tbl.at[pl.ds(c * _CHUNK, _CHUNK)])

        return carry

    lax.fori_loop(0, iters, stage, 0)
    plsc.subcore_barrier()

    ib = (ib0, ib1, ib2, ib3)
    db = (db0, db1, db2, db3)
    isem = (i0, i1, i2, i3)
    gsem = (g0, g1, g2, g3)
    ssem = (s0, s1, s2, s3)
    nset = 4
    niter = nchunk // nset
    base = sid * ept_dir

    def drain_store(k):
        pltpu.make_async_copy(
            db[k], out_hbm.at[cid, pl.ds(base, _CHUNK)], ssem[k]).wait()

    def body(i, carry):
        his = []
        for k in range(nset):
            c = nset * i + k

            @pl.when(i > 0)
            def _(k=k):
                drain_store(k)

            his.append(pltpu.async_copy(
                idx_hbm.at[cid, sid, c], ib[k], isem[k]))
        hgs = []
        for k in range(nset):
            his[k].wait()
            hgs.append(pltpu.async_copy(shared_tbl.at[ib[k]], db[k], gsem[k]))
        for k in range(nset):
            c = nset * i + k
            hgs[k].wait()
            pltpu.async_copy(
                db[k], out_hbm.at[cid, pl.ds(base + c * _CHUNK, _CHUNK)],
                ssem[k])
        return carry

    lax.fori_loop(0, niter, body, 0)
    for k in range(nset):
        drain_store(k)
    for c in range(niter * nset, nchunk):
        pltpu.sync_copy(idx_hbm.at[cid, sid, c], ib[0])
        hg = pltpu.async_copy(shared_tbl.at[ib[0]], db[0], g0)
        hg.wait()
        pltpu.sync_copy(
            db[0], out_hbm.at[cid, pl.ds(base + c * _CHUNK, _CHUNK)])


def _mlp_body(fc, neb, pi_ref, pj_ref, ea_ref, w1c_ref, w3c_ref, w2_ref,
              w4_ref, b1_ref, b2_ref, b3_ref, b4_ref, z_ref, m_ref, zs_ref,
              zq_ref, acc_s, acc_q):
    i = pl.program_id(0)
    bf = jnp.bfloat16
    f32 = jnp.float32
    pi1, pi3 = _unpack2(pi_ref[0])
    pj1, pj3 = _unpack2(pj_ref[0])
    ea = ea_ref[...].astype(bf)
    w2 = w2_ref[...].astype(bf)
    w4 = w4_ref[...].astype(bf)

    a1 = (pi1 + pj1
          + jnp.dot(ea, w1c_ref[...].astype(bf), preferred_element_type=f32)
          + b1_ref[...])
    z = jnp.dot(_silu(a1).astype(bf), w2, preferred_element_type=f32) \
        + b2_ref[...]
    a3 = (pi3 + pj3
          + jnp.dot(ea, w3c_ref[...].astype(bf), preferred_element_type=f32)
          + b3_ref[...])
    m = jnp.dot(_silu(a3).astype(bf), w4, preferred_element_type=f32) \
        + b4_ref[...]

    z_ref[...] = z.astype(bf)
    m_ref[...] = m.astype(bf)

    @pl.when(i == 0)
    def _():
        acc_s[...] = jnp.zeros_like(acc_s)
        acc_q[...] = jnp.zeros_like(acc_q)

    acc_s[...] += jnp.sum(z, axis=0, keepdims=True)
    acc_q[...] += jnp.sum(z * z, axis=0, keepdims=True)

    @pl.when(i == neb - 1)
    def _():
        zs_ref[...] = acc_s[...]
        zq_ref[...] = acc_q[...]


def _score_body(e_f, z_ref, m_ref, zs_ref, zq_ref, g_ref, b_ref, msg_ref):
    mu = zs_ref[...] / e_f
    var = zq_ref[...] / e_f - mu * mu
    rstd = lax.rsqrt(var + _EPS)
    z = z_ref[...].astype(jnp.float32)
    zn = (z - mu) * (rstd * g_ref[...]) + b_ref[...]
    score = jax.nn.sigmoid(zn)
    msg_ref[...] = score * m_ref[...].astype(jnp.float32)


def _scatter_body(n, fc, nchunk, ept, rch, msg_hbm, idx_hbm, out_hbm,
                  shared, vbuf, idxs, m0, m1, l0, l1, t0, t1):
    cid = lax.axis_index("c")
    sid = lax.axis_index("s")
    wid = cid * _NS + sid
    nzc = n // rch
    iters = (nzc + _NS - 1) // _NS

    zero16 = jnp.zeros((16,), jnp.float32)
    lanes = fc // 16

    def zb(k, carry):
        vbuf[k // lanes, pl.ds((k % lanes) * 16, 16)] = zero16
        return carry

    lax.fori_loop(0, rch * lanes, zb, 0)

    def zcopy(k, carry):
        c = k * _NS + sid

        @pl.when(c < nzc)
        def _():
            pltpu.sync_copy(vbuf, shared.at[pl.ds(c * rch, rch)])

        return carry

    lax.fori_loop(0, iters, zcopy, 0)
    plsc.subcore_barrier()

    pltpu.sync_copy(idx_hbm.at[wid], idxs)
    ebase = wid * ept
    mb = (m0, m1)
    ls = (l0, l1)
    ts = (t0, t1)
    nset = 2
    niter = nchunk // nset

    def drain_scatter(k):
        pltpu.make_async_copy(mb[k], shared.at[idxs.at[0]], ts[k]).wait()

    def sb(i, carry):
        hs = []
        for k in range(nset):
            c = nset * i + k

            @pl.when(i > 0)
            def _(k=k):
                drain_scatter(k)

            hs.append(pltpu.async_copy(
                msg_hbm.at[pl.ds(ebase + c * _CHUNK, _CHUNK)], mb[k], ls[k]))
        for k in range(nset):
            c = nset * i + k
            hs[k].wait()
            pltpu.async_copy(mb[k], shared.at[idxs.at[c]], ts[k], add=True)
        return carry

    lax.fori_loop(0, niter, sb, 0)
    for k in range(nset):
        drain_scatter(k)
    for c in range(niter * nset, nchunk):
        pltpu.sync_copy(msg_hbm.at[pl.ds(ebase + c * _CHUNK, _CHUNK)], mb[0])
        pltpu.sync_copy(mb[0], shared.at[idxs.at[c]], add=True)
    plsc.subcore_barrier()

    def ob(k, carry):
        c = k * _NS + sid

        @pl.when(c < nzc)
        def _():
            pltpu.sync_copy(shared.at[pl.ds(c * rch, rch)], vbuf)
            pltpu.sync_copy(vbuf, out_hbm.at[cid, pl.ds(c * rch, rch)])

        return carry

    lax.fori_loop(0, iters, ob, 0)


def _final_body(n_f, p0_ref, p1_ref, x_ref, g_ref, b_ref, y_ref,
                acc_s, acc_q):
    ph = pl.program_id(0)
    i = pl.program_id(1)
    o = p0_ref[0] + p1_ref[0]

    @pl.when(ph == 0)
    def _():
        @pl.when(i == 0)
        def _():
            acc_s[...] = jnp.zeros_like(acc_s)
            acc_q[...] = jnp.zeros_like(acc_q)

        acc_s[...] += jnp.sum(o, axis=0, keepdims=True)
        acc_q[...] += jnp.sum(o * o, axis=0, keepdims=True)

    @pl.when(ph == 1)
    def _():
        mu = acc_s[...] / n_f
        var = acc_q[...] / n_f - mu * mu
        rstd = lax.rsqrt(var + _EPS)
        y_ref[...] = jnp.maximum(
            x_ref[...] + (o - mu) * (rstd * g_ref[...]) + b_ref[...], 0.0)


def kernel(x, edge_index, edge_attr, W1, b1, W2, b2, W3, b3, W4, b4,
           g_int, b_int, g_bn, b_bn):
    f32 = jnp.float32
    bf = jnp.bfloat16
    n, fc = x.shape
    e = edge_index.shape[1]
    nw = _NC * _NS
    ept = e // nw
    nchunk = ept // _CHUNK
    ept_dir = e // _NS
    nchunk_dir = ept_dir // _CHUNK
    eb = 8000
    neb = e // eb
    nb = 2000
    nb0 = 5000
    rch = 80

    src = edge_index[0]
    dst = edge_index[1]
    idx_g = jnp.stack([src, dst]).reshape(2, _NS, nchunk_dir, _CHUNK)
    dst3 = dst.reshape(nw, nchunk, _CHUNK)

    tbl_a, tbl_b = pl.pallas_call(
        functools.partial(_proj_body, fc),
        grid=(n // nb0,),
        in_specs=[
            pl.BlockSpec((nb0, fc), lambda i: (i, 0)),
            pl.BlockSpec((2 * fc, fc), lambda i: (0, 0)),
            pl.BlockSpec((2 * fc, fc), lambda i: (0, 0)),
        ],
        out_specs=[
            pl.BlockSpec((nb0, fc), lambda i: (i, 0)),
            pl.BlockSpec((nb0, fc), lambda i: (i, 0)),
        ],
        out_shape=[
            jax.ShapeDtypeStruct((n, fc), jnp.int32),
            jax.ShapeDtypeStruct((n, fc), jnp.int32),
        ],
    )(x, W1[:2 * fc], W3[:2 * fc])

    mesh = plsc.VectorSubcoreMesh(core_axis_name="c", subcore_axis_name="s")

    gathered = pl.kernel(
        functools.partial(_gather_body, n, fc, nchunk_dir, ept_dir),
        out_type=jax.ShapeDtypeStruct((2, e, fc), jnp.int32),
        mesh=mesh,
        scratch_types=(
            [pltpu.VMEM_SHARED((n, fc), jnp.int32)]
            + [pltpu.VMEM((_CHUNK,), jnp.int32) for _ in range(4)]
            + [pltpu.VMEM((_CHUNK, fc), jnp.int32) for _ in range(4)]
            + [pltpu.SemaphoreType.DMA for _ in range(12)]
        ),
    )(tbl_a, tbl_b, idx_g)

    z, m, zsum, zsq = pl.pallas_call(
        functools.partial(_mlp_body, fc, neb),
        grid=(neb,),
        in_specs=[
            pl.BlockSpec((1, eb, fc), lambda i: (1, i, 0)),
            pl.BlockSpec((1, eb, fc), lambda i: (0, i, 0)),
            pl.BlockSpec((eb, fc), lambda i: (i, 0)),
            pl.BlockSpec((fc, fc), lambda i: (0, 0)),
            pl.BlockSpec((fc, fc), lambda i: (0, 0)),
            pl.BlockSpec((fc, fc), lambda i: (0, 0)),
            pl.BlockSpec((fc, fc), lambda i: (0, 0)),
            pl.BlockSpec((1, fc), lambda i: (0, 0)),
            pl.BlockSpec((1, fc), lambda i: (0, 0)),
            pl.BlockSpec((1, fc), lambda i: (0, 0)),
            pl.BlockSpec((1, fc), lambda i: (0, 0)),
        ],
        out_specs=[
            pl.BlockSpec((eb, fc), lambda i: (i, 0)),
            pl.BlockSpec((eb, fc), lambda i: (i, 0)),
            pl.BlockSpec((1, fc), lambda i: (0, 0)),
            pl.BlockSpec((1, fc), lambda i: (0, 0)),
        ],
        out_shape=[
            jax.ShapeDtypeStruct((e, fc), bf),
            jax.ShapeDtypeStruct((e, fc), bf),
            jax.ShapeDtypeStruct((1, fc), f32),
            jax.ShapeDtypeStruct((1, fc), f32),
        ],
        scratch_shapes=[pltpu.VMEM((1, fc), f32), pltpu.VMEM((1, fc), f32)],
    )(gathered, gathered, edge_attr, W1[2 * fc:], W3[2 * fc:], W2, W4,
      b1.reshape(1, fc), b2.reshape(1, fc), b3.reshape(1, fc),
      b4.reshape(1, fc))

    msg = pl.pallas_call(
        functools.partial(_score_body, float(e)),
        grid=(neb,),
        in_specs=[
            pl.BlockSpec((eb, fc), lambda i: (i, 0)),
            pl.BlockSpec((eb, fc), lambda i: (i, 0)),
            pl.BlockSpec((1, fc), lambda i: (0, 0)),
            pl.BlockSpec((1, fc), lambda i: (0, 0)),
            pl.BlockSpec((1, fc), lambda i: (0, 0)),
            pl.BlockSpec((1, fc), lambda i: (0, 0)),
        ],
        out_specs=pl.BlockSpec((eb, fc), lambda i: (i, 0)),
        out_shape=jax.ShapeDtypeStruct((e, fc), f32),
    )(z, m, zsum, zsq, g_int.reshape(1, fc), b_int.reshape(1, fc))

    partials = pl.kernel(
        functools.partial(_scatter_body, n, fc, nchunk, ept, rch),
        out_type=jax.ShapeDtypeStruct((_NC, n, fc), f32),
        mesh=mesh,
        scratch_types=(
            [pltpu.VMEM_SHARED((n, fc), f32),
             pltpu.VMEM((rch, fc), f32),
             pltpu.VMEM((nchunk, _CHUNK), jnp.int32)]
            + [pltpu.VMEM((_CHUNK, fc), f32) for _ in range(2)]
            + [pltpu.SemaphoreType.DMA for _ in range(4)]
        ),
    )(msg, dst3)

    y = pl.pallas_call(
        functools.partial(_final_body, float(n)),
        grid=(2, n // nb),
        in_specs=[
            pl.BlockSpec((1, nb, fc), lambda p, i: (0, i, 0)),
            pl.BlockSpec((1, nb, fc), lambda p, i: (1, i, 0)),
            pl.BlockSpec((nb, fc), lambda p, i: (i, 0)),
            pl.BlockSpec((1, fc), lambda p, i: (0, 0)),
            pl.BlockSpec((1, fc), lambda p, i: (0, 0)),
        ],
        out_specs=pl.BlockSpec((nb, fc), lambda p, i: (i, 0)),
        out_shape=jax.ShapeDtypeStruct((n, fc), f32),
        scratch_shapes=[pltpu.VMEM((1, fc), f32), pltpu.VMEM((1, fc), f32)],
    )(partials, partials, x, g_bn.reshape(1, fc), b_bn.reshape(1, fc))

    return y
